# Initial kernel scaffold; baseline (speedup 1.0000x reference)
import jax
import jax.numpy as jnp
from jax.experimental import pallas as pl


def kernel(x, h, d, T, b):
    raise NotImplementedError("write your pallas kernel here")


# speedup vs baseline: 627.7813x; 627.7813x over previous
import functools

import jax
import jax.numpy as jnp
from jax import lax
from jax.experimental import pallas as pl
from jax.experimental.pallas import tpu as pltpu
from jax.experimental.pallas import tpu_sc as plsc

NC = 2
NS = 16
NW = NC * NS
LANES = 16
CHUNK = 8192


def _table_body(h_ref, d_ref, T_ref, b_ref, hg_ref, L_ref):
    h = h_ref[...]
    d = d_ref[...]
    T = T_ref[...]
    b = b_ref[0, 0]
    kc = h.shape[1] - 1
    acc = d[0, 1] * (h[:, 0:1] >= T[0, 1]).astype(jnp.float32)
    for t in range(2, kc + 1):
        acc = acc + d[0, t] * (h[:, t:t + 1] >= T[0, t]).astype(jnp.float32)
    L_ref[...] = acc - b
    hg_ref[...] = h[:, 0:1]


def _build_tables(h, d, T, b):
    numh = h.shape[0]
    hg, L = pl.pallas_call(
        _table_body,
        out_shape=[
            jax.ShapeDtypeStruct((numh, 1), jnp.float32),
            jax.ShapeDtypeStruct((numh, 1), jnp.float32),
        ],
    )(h, d.reshape(1, -1), T.reshape(1, -1), jnp.reshape(b, (1, 1)))
    return hg.reshape(numh), L.reshape(numh)


def _sc_body(numh, nchunk, x_hbm, hg_hbm, L_hbm, out_hbm, hg_v, L_v, xb, ob):
    wid = lax.axis_index("s") * NC + lax.axis_index("c")
    base = wid * (nchunk * CHUNK)
    pltpu.sync_copy(hg_hbm, hg_v)
    pltpu.sync_copy(L_hbm, L_v)

    def chunk_body(ci, carry):
        off = base + ci * CHUNK
        pltpu.sync_copy(x_hbm.at[pl.ds(off, CHUNK)], xb)

        def vec_body(i, c2):
            s = pl.multiple_of(i * LANES, LANES)
            xv = xb[pl.ds(s, LANES)]
            lo = jnp.zeros((LANES,), jnp.int32)
            hi = jnp.full((LANES,), numh, jnp.int32)
            for _ in range(numh.bit_length() - 1):
                mid = (lo + hi) >> 1
                hv = plsc.load_gather(hg_v, [mid])
                pred = hv < xv
                lo = jnp.where(pred, mid + 1, lo)
                hi = jnp.where(pred, hi, mid)
            idx = jnp.clip(lo, 1, numh - 1)
            left = plsc.load_gather(hg_v, [idx - 1])
            right = plsc.load_gather(hg_v, [idx])
            pick_left = jnp.abs(xv - left) < jnp.abs(xv - right)
            nidx = jnp.where(pick_left, idx - 1, idx)
            ob[pl.ds(s, LANES)] = plsc.load_gather(L_v, [nidx])
            return c2

        lax.fori_loop(0, CHUNK // LANES, vec_body, 0, unroll=2)
        pltpu.sync_copy(ob, out_hbm.at[pl.ds(off, CHUNK)])
        return carry

    lax.fori_loop(0, nchunk, chunk_body, 0)


def _sc_lookup(xf, hg, L):
    n = xf.shape[0]
    numh = hg.shape[0]
    assert n % (NW * CHUNK) == 0, n
    nchunk = n // (NW * CHUNK)
    mesh = plsc.VectorSubcoreMesh(core_axis_name="c", subcore_axis_name="s")
    return pl.kernel(
        functools.partial(_sc_body, numh, nchunk),
        out_type=jax.ShapeDtypeStruct((n,), jnp.float32),
        mesh=mesh,
        compiler_params=pltpu.CompilerParams(
            needs_layout_passes=False, use_tc_tiling_on_sc=False),
        scratch_types=[
            pltpu.VMEM((numh,), jnp.float32),
            pltpu.VMEM((numh,), jnp.float32),
            pltpu.VMEM((CHUNK,), jnp.float32),
            pltpu.VMEM((CHUNK,), jnp.float32),
        ],
    )(xf, hg, L)


def kernel(x, h, d, T, b):
    sp = x.shape
    hg, L = _build_tables(h, d, T, b)
    out = _sc_lookup(x.reshape(-1), hg, L)
    return out.reshape(sp)


# speedup vs baseline: 641.5057x; 1.0219x over previous
import functools

import jax
import jax.numpy as jnp
from jax import lax
from jax.experimental import pallas as pl
from jax.experimental.pallas import tpu as pltpu
from jax.experimental.pallas import tpu_sc as plsc

NC = 2
NS = 16
NW = NC * NS
LANES = 16
CHUNK = 16384


def _table_body(h_ref, d_ref, T_ref, b_ref, hg_ref, L_ref):
    h = h_ref[...]
    d = d_ref[...]
    T = T_ref[...]
    b = b_ref[0, 0]
    kc = h.shape[1] - 1
    acc = d[0, 1] * (h[:, 0:1] >= T[0, 1]).astype(jnp.float32)
    for t in range(2, kc + 1):
        acc = acc + d[0, t] * (h[:, t:t + 1] >= T[0, t]).astype(jnp.float32)
    L_ref[...] = acc - b
    hg_ref[...] = h[:, 0:1]


def _build_tables(h, d, T, b):
    numh = h.shape[0]
    hg, L = pl.pallas_call(
        _table_body,
        out_shape=[
            jax.ShapeDtypeStruct((numh, 1), jnp.float32),
            jax.ShapeDtypeStruct((numh, 1), jnp.float32),
        ],
    )(h, d.reshape(1, -1), T.reshape(1, -1), jnp.reshape(b, (1, 1)))
    return hg.reshape(numh), L.reshape(numh)


def _sc_body(numh, nchunk, x_hbm, hg_hbm, L_hbm, out_hbm, hg_v, L_v, xb, ob):
    wid = lax.axis_index("s") * NC + lax.axis_index("c")
    base = wid * (nchunk * CHUNK)
    pltpu.sync_copy(hg_hbm, hg_v)
    pltpu.sync_copy(L_hbm, L_v)

    def chunk_body(ci, carry):
        off = base + ci * CHUNK
        pltpu.sync_copy(x_hbm.at[pl.ds(off, CHUNK)], xb)

        def vec_body(i, c2):
            s = pl.multiple_of(i * LANES, LANES)
            xv = xb[pl.ds(s, LANES)]
            pos = jnp.zeros((LANES,), jnp.int32)
            stp = numh >> 1
            while stp > 0:
                hv = plsc.load_gather(hg_v, [pos + (stp - 1)])
                pos = pos + jnp.where(hv < xv, stp, 0)
                stp >>= 1
            idx = jnp.clip(pos, 1, numh - 1)
            left = plsc.load_gather(hg_v, [idx - 1])
            right = plsc.load_gather(hg_v, [idx])
            pick_left = jnp.abs(xv - left) < jnp.abs(xv - right)
            nidx = jnp.where(pick_left, idx - 1, idx)
            ob[pl.ds(s, LANES)] = plsc.load_gather(L_v, [nidx])
            return c2

        lax.fori_loop(0, CHUNK // LANES, vec_body, 0, unroll=4)
        pltpu.sync_copy(ob, out_hbm.at[pl.ds(off, CHUNK)])
        return carry

    lax.fori_loop(0, nchunk, chunk_body, 0)


def _sc_lookup(xf, hg, L):
    n = xf.shape[0]
    numh = hg.shape[0]
    assert n % (NW * CHUNK) == 0, n
    nchunk = n // (NW * CHUNK)
    mesh = plsc.VectorSubcoreMesh(core_axis_name="c", subcore_axis_name="s")
    return pl.kernel(
        functools.partial(_sc_body, numh, nchunk),
        out_type=jax.ShapeDtypeStruct((n,), jnp.float32),
        mesh=mesh,
        compiler_params=pltpu.CompilerParams(
            needs_layout_passes=False, use_tc_tiling_on_sc=False),
        scratch_types=[
            pltpu.VMEM((numh,), jnp.float32),
            pltpu.VMEM((numh,), jnp.float32),
            pltpu.VMEM((CHUNK,), jnp.float32),
            pltpu.VMEM((CHUNK,), jnp.float32),
        ],
    )(xf, hg, L)


def kernel(x, h, d, T, b):
    sp = x.shape
    hg, L = _build_tables(h, d, T, b)
    out = _sc_lookup(x.reshape(-1), hg, L)
    return out.reshape(sp)
